# baseline (speedup 1.0000x reference)
import functools

import jax
import jax.numpy as jnp
from jax import lax
from jax.experimental import pallas as pl
from jax.experimental.pallas import tpu as pltpu
from jax.experimental.pallas import tpu_sc as plsc

BATCH = 16384
EMBED = 64
NC = 2
NS = 16
NW = NC * NS
B_PER_W = BATCH // NW
CHUNK = 128
NCHUNK = B_PER_W // CHUNK


def _gather_body(uidx_hbm, bidx_hbm, utab_hbm, btab_hbm, u_out, v_out,
                 idx_u, idx_b, rows_u, rows_b, sem):
    wid = lax.axis_index("s") * NC + lax.axis_index("c")
    base = wid * B_PER_W
    pltpu.sync_copy(uidx_hbm.at[wid], idx_u)
    pltpu.sync_copy(bidx_hbm.at[wid], idx_b)
    copies = []
    for j in range(NCHUNK):
        copies.append(pltpu.async_copy(
            utab_hbm.at[idx_u.at[j]], rows_u.at[pl.ds(j * CHUNK, CHUNK)], sem))
        copies.append(pltpu.async_copy(
            btab_hbm.at[idx_b.at[j]], rows_b.at[pl.ds(j * CHUNK, CHUNK)], sem))
    for c in copies:
        c.wait()
    pltpu.sync_copy(rows_u, u_out.at[pl.ds(base, B_PER_W)])
    pltpu.sync_copy(rows_b, v_out.at[pl.ds(base, B_PER_W)])


@functools.cache
def _make_gather():
    return functools.partial(
        pl.kernel,
        mesh=plsc.VectorSubcoreMesh(core_axis_name="c", subcore_axis_name="s"),
        compiler_params=pltpu.CompilerParams(use_tc_tiling_on_sc=False),
        out_type=[
            jax.ShapeDtypeStruct((BATCH, EMBED), jnp.float32),
            jax.ShapeDtypeStruct((BATCH, EMBED), jnp.float32),
        ],
        scratch_types=[
            pltpu.VMEM((NCHUNK, CHUNK), jnp.int32),
            pltpu.VMEM((NCHUNK, CHUNK), jnp.int32),
            pltpu.VMEM((B_PER_W, EMBED), jnp.float32),
            pltpu.VMEM((B_PER_W, EMBED), jnp.float32),
            pltpu.SemaphoreType.DMA,
        ],
    )(_gather_body)


def _bn_relu(h, g, be, eps=1e-5):
    mean = jnp.mean(h, axis=0, keepdims=True)
    c = h - mean
    var = jnp.mean(c * c, axis=0, keepdims=True)
    return jnp.maximum(c * lax.rsqrt(var + eps) * g + be, 0.0)


def _mlp_body(u_ref, v_ref, w1a_ref, w1b_ref, b1_ref, g1_ref, be1_ref,
              w2_ref, b2_ref, g2_ref, be2_ref, w3_ref, b3_ref, out_ref):
    h = (jnp.dot(u_ref[...], w1a_ref[...], preferred_element_type=jnp.float32)
         + jnp.dot(v_ref[...], w1b_ref[...], preferred_element_type=jnp.float32)
         + b1_ref[...])
    h = _bn_relu(h, g1_ref[...], be1_ref[...])
    h2 = jnp.dot(h, w2_ref[...], preferred_element_type=jnp.float32) + b2_ref[...]
    h2 = _bn_relu(h2, g2_ref[...], be2_ref[...])
    out_ref[...] = (jnp.dot(h2, w3_ref[...], preferred_element_type=jnp.float32)
                    + b3_ref[...])


_mlp = pl.pallas_call(
    _mlp_body,
    out_shape=jax.ShapeDtypeStruct((BATCH, 1), jnp.float32),
)


def kernel(user_input, book_input, user_table, book_table,
           W1, b1, g1, be1, W2, b2, g2, be2, W3, b3):
    uidx = user_input.reshape(NW, NCHUNK, CHUNK)
    bidx = book_input.reshape(NW, NCHUNK, CHUNK)
    u_rows, v_rows = _make_gather()(uidx, bidx, user_table, book_table)
    out = _mlp(u_rows, v_rows, W1[:EMBED], W1[EMBED:],
               b1.reshape(1, -1), g1.reshape(1, -1), be1.reshape(1, -1),
               W2, b2.reshape(1, -1), g2.reshape(1, -1), be2.reshape(1, -1),
               W3, b3.reshape(1, 1))
    return out.reshape(BATCH)
